# baseline (speedup 1.0000x reference)
import functools

import jax
import jax.numpy as jnp
from jax import lax
from jax.experimental import pallas as pl
from jax.experimental.pallas import tpu as pltpu
from jax.experimental.pallas import tpu_sc as plsc

B = 16
N = 256
N1 = N + 1
H = 16
NUM_EDGES = 16384
NUM_SPATIAL = 512
EDGE_DIM = 4
NE = N * EDGE_DIM
NC, NS, L = 2, 16, 16
NW = NC * NS
TASKS = B * N
TPW = TASKS // NW
K = 4
CPT = TPW // K
KN = K * N


def _sc_bias(ctable, sp1, et1, *, interpret=False):
    mesh = plsc.VectorSubcoreMesh(
        core_axis_name="c", subcore_axis_name="s",
        num_cores=NC, num_subcores=NS)

    scr = dict(
        sp=pltpu.VMEM((KN,), jnp.int32),
        et=pltpu.VMEM((K * NE,), jnp.int32),
        idxA=pltpu.VMEM((EDGE_DIM, KN), jnp.int32),
        idxB=pltpu.VMEM((EDGE_DIM, KN), jnp.int32),
        biasA=pltpu.VMEM((KN, H), jnp.float32),
        biasB=pltpu.VMEM((KN, H), jnp.float32),
        stgA=pltpu.VMEM((H * K, N), jnp.float32),
        stgB=pltpu.VMEM((H * K, N), jnp.float32),
        dstA=pltpu.VMEM((H * K,), jnp.int32), dstB=pltpu.VMEM((H * K,), jnp.int32),
        isem=pltpu.SemaphoreType.DMA,
        gsemA=pltpu.SemaphoreType.DMA, gsemB=pltpu.SemaphoreType.DMA,
        ssemA=pltpu.SemaphoreType.DMA, ssemB=pltpu.SemaphoreType.DMA,
    )

    @functools.partial(
        pl.kernel,
        out_type=jax.ShapeDtypeStruct((B * H * N, N), jnp.float32),
        mesh=mesh,
        scratch_types=scr,
        compiler_params=pltpu.CompilerParams(
            needs_layout_passes=False, use_tc_tiling_on_sc=False),
        interpret=interpret,
    )
    def k(ct_hbm, sp_hbm, et_hbm, out_hbm, *, sp, et, idxA, idxB,
          biasA, biasB, stgA, stgB, dstA, dstB,
          isem, gsemA, gsemB, ssemA, ssemB):
        wid = lax.axis_index("c") * NS + lax.axis_index("s")
        base = wid * TPW
        iota = lax.iota(jnp.int32, L)
        slots = (
            dict(idx=idxA, bias=biasA, stg=stgA, dst=dstA,
                 gsem=gsemA, ssem=ssemA),
            dict(idx=idxB, bias=biasB, stg=stgB, dst=dstB,
                 gsem=gsemB, ssem=ssemB),
        )

        def issue_idx_fetch(chunk):
            r0 = base + chunk * K
            pltpu.async_copy(sp_hbm.at[pl.ds(r0 * N, KN)], sp, isem)
            pltpu.async_copy(et_hbm.at[pl.ds(r0 * NE, K * NE)], et, isem)

        def transpose_tasks(ot, lo, hi):
            def tk(kk, _):
                for cc in range(N // L):
                    rowv = kk * N + cc * L + iota
                    for h in range(H):
                        hv = jnp.full((L,), h, jnp.int32)
                        vals = plsc.load_gather(ot["bias"], [rowv, hv])
                        ot["stg"][h * K + kk, pl.ds(cc * L, L)] = vals
                return ()
            lax.fori_loop(lo, hi, tk, ())

        def half_iter(c, par):
            me, ot = slots[par], slots[1 - par]
            in_front = c < CPT
            in_back = jnp.logical_and(c >= 1, c <= CPT)

            @pl.when(in_front)
            def _front():
                r0 = base + c * K
                pltpu.make_async_copy(sp_hbm.at[pl.ds(r0 * N, KN)], sp,
                                      isem).wait()
                pltpu.make_async_copy(et_hbm.at[pl.ds(r0 * NE, K * NE)],
                                      et, isem).wait()

                def dk(kk, _):
                    for d in range(EDGE_DIM):
                        for cc in range(16):
                            vals = plsc.load_gather(
                                et,
                                [iota * EDGE_DIM + (kk * NE + cc * 64 + d)])
                            me["idx"][d, pl.ds(kk * N + cc * L, L)] = (
                                vals + (NUM_SPATIAL + 1))
                    return ()
                lax.fori_loop(0, K, dk, ())

            @pl.when(in_back)
            def _drain_prev():
                for d in range(EDGE_DIM):
                    pltpu.make_async_copy(ct_hbm.at[ot["idx"].at[d]],
                                          ot["bias"], ot["gsem"]).wait()
                @pl.when(c >= 3)
                def _():
                    pltpu.make_async_copy(ot["stg"], out_hbm.at[ot["dst"]],
                                          ot["ssem"]).wait()

            @pl.when(in_front)
            def _spatial():
                pltpu.async_copy(ct_hbm.at[sp], me["bias"], me["gsem"])

            @pl.when(in_back)
            def _back1():
                transpose_tasks(ot, 0, K // 2)

            @pl.when(in_front)
            def _edges():
                pltpu.make_async_copy(ct_hbm.at[sp], me["bias"],
                                      me["gsem"]).wait()
                for d in range(EDGE_DIM):
                    pltpu.async_copy(ct_hbm.at[me["idx"].at[d]], me["bias"],
                                     me["gsem"], add=True)

                @pl.when(c + 1 < CPT)
                def _():
                    issue_idx_fetch(c + 1)

            @pl.when(in_back)
            def _back2():
                transpose_tasks(ot, K // 2, K)
                r0 = base + (c - 1) * K
                bb = r0 // N
                i0 = r0 - bb * N
                cbase = bb * (H * N) + i0
                for kk in range(K):
                    vec = iota * N + (cbase + kk)
                    plsc.store_scatter(ot["dst"], [iota * K + kk], vec)
                pltpu.async_copy(ot["stg"], out_hbm.at[ot["dst"]], ot["ssem"])

        issue_idx_fetch(0)

        def pair(c2, _):
            half_iter(2 * c2, 0)
            half_iter(2 * c2 + 1, 1)
            return ()
        lax.fori_loop(0, CPT // 2 + 1, pair, ())

        for sl in slots:
            pltpu.make_async_copy(sl["stg"], out_hbm.at[sl["dst"]],
                                  sl["ssem"]).wait()

    return k(ctable, sp1, et1)


def _asm_body(ab_ref, e_ref, t_ref, o_ref):
    h = pl.program_id(1)
    tv = t_ref[0, h]
    ab2 = ab_ref[0] * 2.0
    e = e_ref[0, 0]
    o_ref[0, 0, 0:1, :] = ab2[0:1, :] + tv
    o_ref[0, 0, 1:, 0:1] = ab2[1:, 0:1] + tv
    o_ref[0, 0, 1:, 1:] = ab2[1:, 1:] + e


def _tc_assemble(ab, et4, t, *, interpret=False):
    return pl.pallas_call(
        _asm_body,
        grid=(B, H),
        in_specs=[
            pl.BlockSpec((1, N1, N1), lambda b, h: (b, 0, 0)),
            pl.BlockSpec((1, 1, N, N), lambda b, h: (b, h, 0, 0)),
            pl.BlockSpec(memory_space=pltpu.SMEM),
        ],
        out_specs=pl.BlockSpec((1, 1, N1, N1), lambda b, h: (b, h, 0, 0)),
        out_shape=jax.ShapeDtypeStruct((B, H, N1, N1), jnp.float32),
        interpret=interpret,
    )(ab, et4, t)


def kernel(attn_bias, spatial_pos, edge_input, attn_edge_type,
           edge_encoder_weight, spatial_pos_encoder_weight,
           graph_token_virtual_distance_weight):
    sw0 = spatial_pos_encoder_weight.at[0].set(0.0)
    ew0 = edge_encoder_weight.at[0].set(0.0) * 0.25
    ctable = jnp.concatenate([sw0, ew0], axis=0)
    sp1 = spatial_pos.reshape(B * N * N)
    et1 = attn_edge_type.reshape(B * N * N * EDGE_DIM)
    et_flat = _sc_bias(ctable, sp1, et1)
    et4 = et_flat.reshape(B, H, N, N)
    return _tc_assemble(attn_bias, et4, graph_token_virtual_distance_weight)


# speedup vs baseline: 2.8410x; 2.8410x over previous
import functools

import jax
import jax.numpy as jnp
from jax import lax
from jax.experimental import pallas as pl
from jax.experimental.pallas import tpu as pltpu
from jax.experimental.pallas import tpu_sc as plsc

B = 16
N = 256
N1 = N + 1
H = 16
NUM_EDGES = 16384
NUM_SPATIAL = 512
EDGE_DIM = 4
NE = N * EDGE_DIM
NC, NS, L = 2, 16, 16
NW = NC * NS
TASKS = B * N
TPW = TASKS // NW
K = 4
CPT = TPW // K
KN = K * N


def _sc_bias(ctable, sp1, et1, *, interpret=False):
    mesh = plsc.VectorSubcoreMesh(
        core_axis_name="c", subcore_axis_name="s",
        num_cores=NC, num_subcores=NS)

    scr = dict(
        sp=pltpu.VMEM((K, N), jnp.int32),
        et=pltpu.VMEM((K, NE), jnp.int32),
        idxA=pltpu.VMEM((EDGE_DIM + 1, KN), jnp.int32),
        idxB=pltpu.VMEM((EDGE_DIM + 1, KN), jnp.int32),
        biasA=pltpu.VMEM((KN, H), jnp.float32),
        biasB=pltpu.VMEM((KN, H), jnp.float32),
        stgA=pltpu.VMEM((H * K, N), jnp.float32),
        stgB=pltpu.VMEM((H * K, N), jnp.float32),
        dstA=pltpu.VMEM((H * K,), jnp.int32), dstB=pltpu.VMEM((H * K,), jnp.int32),
        isem=pltpu.SemaphoreType.DMA,
        gsemA=pltpu.SemaphoreType.DMA, gsemB=pltpu.SemaphoreType.DMA,
        ssemA=pltpu.SemaphoreType.DMA, ssemB=pltpu.SemaphoreType.DMA,
    )

    @functools.partial(
        pl.kernel,
        out_type=jax.ShapeDtypeStruct((B * H * N, N), jnp.float32),
        mesh=mesh,
        scratch_types=scr,
        compiler_params=pltpu.CompilerParams(
            needs_layout_passes=False, use_tc_tiling_on_sc=False),
        interpret=interpret,
    )
    def k(ct_hbm, sp_hbm, et_hbm, out_hbm, *, sp, et, idxA, idxB,
          biasA, biasB, stgA, stgB, dstA, dstB,
          isem, gsemA, gsemB, ssemA, ssemB):
        wid = lax.axis_index("c") * NS + lax.axis_index("s")
        base = wid * TPW
        iota = lax.iota(jnp.int32, L)
        slots = (
            dict(idx=idxA, bias=biasA, stg=stgA, dst=dstA,
                 gsem=gsemA, ssem=ssemA),
            dict(idx=idxB, bias=biasB, stg=stgB, dst=dstB,
                 gsem=gsemB, ssem=ssemB),
        )

        def issue_idx_fetch(chunk):
            r0 = base + chunk * K
            pltpu.async_copy(sp_hbm.at[pl.ds(r0, K)], sp, isem)
            pltpu.async_copy(et_hbm.at[pl.ds(r0, K)], et, isem)

        def transpose_tasks(ot, lo, hi):
            def tk(kk, _):
                for cc in range(N // L):
                    rowv = kk * N + cc * L + iota
                    for h in range(H):
                        hv = jnp.full((L,), h, jnp.int32)
                        vals = plsc.load_gather(ot["bias"], [rowv, hv])
                        ot["stg"][h * K + kk, pl.ds(cc * L, L)] = vals
                return ()
            lax.fori_loop(lo, hi, tk, ())

        def half_iter(c, par):
            me, ot = slots[par], slots[1 - par]
            in_front = c < CPT
            in_back = jnp.logical_and(c >= 1, c <= CPT)

            @pl.when(in_front)
            def _front():
                r0 = base + c * K
                pltpu.make_async_copy(sp_hbm.at[pl.ds(r0, K)], sp,
                                      isem).wait()
                pltpu.make_async_copy(et_hbm.at[pl.ds(r0, K)], et,
                                      isem).wait()

                def dk(kk, _):
                    for d in range(EDGE_DIM):
                        for cc in range(16):
                            vals = plsc.load_gather(
                                et.at[kk],
                                [iota * EDGE_DIM + (cc * 64 + d)])
                            me["idx"][d, pl.ds(kk * N + cc * L, L)] = (
                                vals + (NUM_SPATIAL + 1))
                    for cc in range(16):
                        me["idx"][EDGE_DIM, pl.ds(kk * N + cc * L, L)] = (
                            sp[kk, pl.ds(cc * L, L)])
                    return ()
                lax.fori_loop(0, K, dk, ())

            @pl.when(in_back)
            def _drain_prev():
                for d in range(EDGE_DIM):
                    pltpu.make_async_copy(ct_hbm.at[ot["idx"].at[d]],
                                          ot["bias"], ot["gsem"]).wait()
                @pl.when(c >= 3)
                def _():
                    pltpu.make_async_copy(ot["stg"], out_hbm.at[ot["dst"]],
                                          ot["ssem"]).wait()

            @pl.when(in_front)
            def _spatial():
                pltpu.async_copy(ct_hbm.at[me["idx"].at[EDGE_DIM]],
                                 me["bias"], me["gsem"])

                @pl.when(c + 1 < CPT)
                def _():
                    issue_idx_fetch(c + 1)

            @pl.when(in_back)
            def _back1():
                transpose_tasks(ot, 0, K // 2)

            @pl.when(in_front)
            def _edges():
                pltpu.make_async_copy(ct_hbm.at[me["idx"].at[EDGE_DIM]],
                                      me["bias"], me["gsem"]).wait()
                for d in range(EDGE_DIM):
                    pltpu.async_copy(ct_hbm.at[me["idx"].at[d]], me["bias"],
                                     me["gsem"], add=True)

            @pl.when(in_back)
            def _back2():
                transpose_tasks(ot, K // 2, K)
                r0 = base + (c - 1) * K
                bb = r0 // N
                i0 = r0 - bb * N
                cbase = bb * (H * N) + i0
                for kk in range(K):
                    vec = iota * N + (cbase + kk)
                    plsc.store_scatter(ot["dst"], [iota * K + kk], vec)
                pltpu.async_copy(ot["stg"], out_hbm.at[ot["dst"]], ot["ssem"])

        issue_idx_fetch(0)

        def pair(c2, _):
            half_iter(2 * c2, 0)
            half_iter(2 * c2 + 1, 1)
            return ()
        lax.fori_loop(0, CPT // 2 + 1, pair, ())

        for sl in slots:
            pltpu.make_async_copy(sl["stg"], out_hbm.at[sl["dst"]],
                                  sl["ssem"]).wait()

    return k(ctable, sp1, et1)


def _asm_body(ab_ref, e_ref, t_ref, o_ref):
    h = pl.program_id(1)
    tv = t_ref[0, h]
    ab2 = ab_ref[0] * 2.0
    e = e_ref[0, 0]
    o_ref[0, 0, 0:1, :] = ab2[0:1, :] + tv
    o_ref[0, 0, 1:, 0:1] = ab2[1:, 0:1] + tv
    o_ref[0, 0, 1:, 1:] = ab2[1:, 1:] + e


def _tc_assemble(ab, et4, t, *, interpret=False):
    return pl.pallas_call(
        _asm_body,
        grid=(B, H),
        in_specs=[
            pl.BlockSpec((1, N1, N1), lambda b, h: (b, 0, 0)),
            pl.BlockSpec((1, 1, N, N), lambda b, h: (b, h, 0, 0)),
            pl.BlockSpec(memory_space=pltpu.SMEM),
        ],
        out_specs=pl.BlockSpec((1, 1, N1, N1), lambda b, h: (b, h, 0, 0)),
        out_shape=jax.ShapeDtypeStruct((B, H, N1, N1), jnp.float32),
        interpret=interpret,
    )(ab, et4, t)


def kernel(attn_bias, spatial_pos, edge_input, attn_edge_type,
           edge_encoder_weight, spatial_pos_encoder_weight,
           graph_token_virtual_distance_weight):
    sw0 = spatial_pos_encoder_weight.at[0].set(0.0)
    ew0 = edge_encoder_weight.at[0].set(0.0) * 0.25
    ctable = jnp.concatenate([sw0, ew0], axis=0)
    sp2 = spatial_pos.reshape(B * N, N)
    et2 = attn_edge_type.reshape(B * N, N * EDGE_DIM)
    et_flat = _sc_bias(ctable, sp2, et2)
    et4 = et_flat.reshape(B, H, N, N)
    return _tc_assemble(attn_bias, et4, graph_token_virtual_distance_weight)


# speedup vs baseline: 3.1377x; 1.1044x over previous
import functools

import jax
import jax.numpy as jnp
from jax import lax
from jax.experimental import pallas as pl
from jax.experimental.pallas import tpu as pltpu
from jax.experimental.pallas import tpu_sc as plsc

B = 16
N = 256
N1 = N + 1
H = 16
NUM_EDGES = 16384
NUM_SPATIAL = 512
EDGE_DIM = 4
NE = N * EDGE_DIM
NC, NS, L = 2, 16, 16
NW = NC * NS
TASKS = B * N
TPW = TASKS // NW
K = 4
CPT = TPW // K
KN = K * N


def _sc_bias(ctable, sp1, et1, *, interpret=False):
    mesh = plsc.VectorSubcoreMesh(
        core_axis_name="c", subcore_axis_name="s",
        num_cores=NC, num_subcores=NS)

    scr = dict(
        sp=pltpu.VMEM((K, N), jnp.int32),
        et=pltpu.VMEM((K, NE), jnp.int32),
        idxA=pltpu.VMEM((EDGE_DIM + 1, KN), jnp.int32),
        idxB=pltpu.VMEM((EDGE_DIM + 1, KN), jnp.int32),
        biasA=pltpu.VMEM((KN, H), jnp.float32),
        biasB=pltpu.VMEM((KN, H), jnp.float32),
        stgA=pltpu.VMEM((H * K * 2, N // 2), jnp.float32),
        stgB=pltpu.VMEM((H * K * 2, N // 2), jnp.float32),
        dstA=pltpu.VMEM((H * K * 2,), jnp.int32),
        dstB=pltpu.VMEM((H * K * 2,), jnp.int32),
        isem=pltpu.SemaphoreType.DMA,
        gsemA=pltpu.SemaphoreType.DMA, gsemB=pltpu.SemaphoreType.DMA,
        ssemA=pltpu.SemaphoreType.DMA, ssemB=pltpu.SemaphoreType.DMA,
    )

    @functools.partial(
        pl.kernel,
        out_type=jax.ShapeDtypeStruct((B * H * N * 2, N // 2), jnp.float32),
        mesh=mesh,
        scratch_types=scr,
        compiler_params=pltpu.CompilerParams(
            needs_layout_passes=False, use_tc_tiling_on_sc=False),
        interpret=interpret,
    )
    def k(ct_hbm, sp_hbm, et_hbm, out_hbm, *, sp, et, idxA, idxB,
          biasA, biasB, stgA, stgB, dstA, dstB,
          isem, gsemA, gsemB, ssemA, ssemB):
        wid = lax.axis_index("c") * NS + lax.axis_index("s")
        base = wid * TPW
        iota = lax.iota(jnp.int32, L)
        slots = (
            dict(idx=idxA, bias=biasA, stg=stgA, dst=dstA,
                 gsem=gsemA, ssem=ssemA),
            dict(idx=idxB, bias=biasB, stg=stgB, dst=dstB,
                 gsem=gsemB, ssem=ssemB),
        )

        def issue_idx_fetch(chunk):
            r0 = base + chunk * K
            pltpu.async_copy(sp_hbm.at[pl.ds(r0, K)], sp, isem)
            pltpu.async_copy(et_hbm.at[pl.ds(r0, K)], et, isem)

        def transpose_tasks(ot, lo, hi):
            def tk(kk, _):
                for cc in range(N // L):
                    rowv = kk * N + cc * L + iota
                    for h in range(H):
                        hv = jnp.full((L,), h, jnp.int32)
                        vals = plsc.load_gather(ot["bias"], [rowv, hv])
                        ot["stg"][2 * (h * K + kk) + cc // 8,
                                  pl.ds((cc % 8) * L, L)] = vals
                return ()
            lax.fori_loop(lo, hi, tk, ())

        def half_iter(c, par):
            me, ot = slots[par], slots[1 - par]
            in_front = c < CPT
            in_back = jnp.logical_and(c >= 1, c <= CPT)

            @pl.when(in_front)
            def _front():
                r0 = base + c * K
                pltpu.make_async_copy(sp_hbm.at[pl.ds(r0, K)], sp,
                                      isem).wait()
                pltpu.make_async_copy(et_hbm.at[pl.ds(r0, K)], et,
                                      isem).wait()

                def dk(kk, _):
                    for d in range(EDGE_DIM):
                        for cc in range(16):
                            vals = plsc.load_gather(
                                et.at[kk],
                                [iota * EDGE_DIM + (cc * 64 + d)])
                            me["idx"][d, pl.ds(kk * N + cc * L, L)] = (
                                vals + (NUM_SPATIAL + 1))
                    for cc in range(16):
                        me["idx"][EDGE_DIM, pl.ds(kk * N + cc * L, L)] = (
                            sp[kk, pl.ds(cc * L, L)])
                    return ()
                lax.fori_loop(0, K, dk, ())

            @pl.when(in_back)
            def _drain_prev():
                for d in range(EDGE_DIM):
                    pltpu.make_async_copy(ct_hbm.at[ot["idx"].at[d]],
                                          ot["bias"], ot["gsem"]).wait()
                @pl.when(c >= 3)
                def _():
                    pltpu.make_async_copy(ot["stg"], out_hbm.at[ot["dst"]],
                                          ot["ssem"]).wait()

            @pl.when(in_front)
            def _spatial():
                pltpu.async_copy(ct_hbm.at[me["idx"].at[EDGE_DIM]],
                                 me["bias"], me["gsem"])

                @pl.when(c + 1 < CPT)
                def _():
                    issue_idx_fetch(c + 1)

            @pl.when(in_back)
            def _back1():
                transpose_tasks(ot, 0, K // 2)

            @pl.when(in_front)
            def _edges():
                pltpu.make_async_copy(ct_hbm.at[me["idx"].at[EDGE_DIM]],
                                      me["bias"], me["gsem"]).wait()
                for d in range(EDGE_DIM):
                    pltpu.async_copy(ct_hbm.at[me["idx"].at[d]], me["bias"],
                                     me["gsem"], add=True)

            @pl.when(in_back)
            def _back2():
                transpose_tasks(ot, K // 2, K)
                r0 = base + (c - 1) * K
                bb = r0 // N
                i0 = r0 - bb * N
                cbase = bb * (H * N) + i0
                for kk in range(K):
                    vec = (iota * N + (cbase + kk)) * 2
                    plsc.store_scatter(ot["dst"], [iota * (2 * K) + 2 * kk],
                                       vec)
                    plsc.store_scatter(ot["dst"],
                                       [iota * (2 * K) + (2 * kk + 1)],
                                       vec + 1)
                pltpu.async_copy(ot["stg"], out_hbm.at[ot["dst"]], ot["ssem"])

        issue_idx_fetch(0)

        def pair(c2, _):
            half_iter(2 * c2, 0)
            half_iter(2 * c2 + 1, 1)
            return ()
        lax.fori_loop(0, CPT // 2 + 1, pair, ())

        for sl in slots:
            pltpu.make_async_copy(sl["stg"], out_hbm.at[sl["dst"]],
                                  sl["ssem"]).wait()

    return k(ctable, sp1, et1)


def _asm_body(ab_ref, e_ref, t_ref, o_ref):
    h = pl.program_id(1)
    tv = t_ref[0, h]
    ab2 = ab_ref[0] * 2.0
    e = e_ref[...].reshape(N, N)
    o_ref[0, 0, 0:1, :] = ab2[0:1, :] + tv
    o_ref[0, 0, 1:, 0:1] = ab2[1:, 0:1] + tv
    o_ref[0, 0, 1:, 1:] = ab2[1:, 1:] + e


def _tc_assemble(ab, et4, t, *, interpret=False):
    return pl.pallas_call(
        _asm_body,
        grid=(B, H),
        in_specs=[
            pl.BlockSpec((1, N1, N1), lambda b, h: (b, 0, 0)),
            pl.BlockSpec((2 * N, N // 2), lambda b, h: (b * H + h, 0)),
            pl.BlockSpec(memory_space=pltpu.SMEM),
        ],
        out_specs=pl.BlockSpec((1, 1, N1, N1), lambda b, h: (b, h, 0, 0)),
        out_shape=jax.ShapeDtypeStruct((B, H, N1, N1), jnp.float32),
        interpret=interpret,
    )(ab, et4, t)


def kernel(attn_bias, spatial_pos, edge_input, attn_edge_type,
           edge_encoder_weight, spatial_pos_encoder_weight,
           graph_token_virtual_distance_weight):
    sw0 = spatial_pos_encoder_weight.at[0].set(0.0)
    ew0 = edge_encoder_weight.at[0].set(0.0) * 0.25
    ctable = jnp.concatenate([sw0, ew0], axis=0)
    sp2 = spatial_pos.reshape(B * N, N)
    et2 = attn_edge_type.reshape(B * N, N * EDGE_DIM)
    et_flat = _sc_bias(ctable, sp2, et2)
    return _tc_assemble(attn_bias, et_flat,
                        graph_token_virtual_distance_weight)


# speedup vs baseline: 3.8299x; 1.2206x over previous
import functools

import jax
import jax.numpy as jnp
from jax import lax
from jax.experimental import pallas as pl
from jax.experimental.pallas import tpu as pltpu
from jax.experimental.pallas import tpu_sc as plsc

B = 16
N = 256
N1 = N + 1
H = 16
NUM_EDGES = 16384
NUM_SPATIAL = 512
EDGE_DIM = 4
NE = N * EDGE_DIM
NC, NS, L = 2, 16, 16
NW = NC * NS
TASKS = B * N
TPW = TASKS // NW
K = 4
CPT = TPW // K
KN = K * N


def _sc_bias(ctable, sp1, et1, *, interpret=False):
    mesh = plsc.VectorSubcoreMesh(
        core_axis_name="c", subcore_axis_name="s",
        num_cores=NC, num_subcores=NS)

    scr = dict(
        sp=pltpu.VMEM((K, N), jnp.int32),
        et=pltpu.VMEM((K, NE), jnp.int32),
        idxA=pltpu.VMEM((EDGE_DIM + 1, KN), jnp.int32),
        idxB=pltpu.VMEM((EDGE_DIM + 1, KN), jnp.int32),
        biasA=pltpu.VMEM((KN, H), jnp.float32),
        biasB=pltpu.VMEM((KN, H), jnp.float32),
        stgA=pltpu.VMEM((H * K * 2, N // 2), jnp.float32),
        stgB=pltpu.VMEM((H * K * 2, N // 2), jnp.float32),
        dstA=pltpu.VMEM((H * K * 2,), jnp.int32),
        dstB=pltpu.VMEM((H * K * 2,), jnp.int32),
        isem=pltpu.SemaphoreType.DMA,
        gsemA=pltpu.SemaphoreType.DMA, gsemB=pltpu.SemaphoreType.DMA,
        ssemA=pltpu.SemaphoreType.DMA, ssemB=pltpu.SemaphoreType.DMA,
    )

    @functools.partial(
        pl.kernel,
        out_type=jax.ShapeDtypeStruct((B * H * N * 2, N // 2), jnp.float32),
        mesh=mesh,
        scratch_types=scr,
        compiler_params=pltpu.CompilerParams(
            needs_layout_passes=False, use_tc_tiling_on_sc=False),
        interpret=interpret,
    )
    def k(ct_hbm, sp_hbm, et_hbm, out_hbm, *, sp, et, idxA, idxB,
          biasA, biasB, stgA, stgB, dstA, dstB,
          isem, gsemA, gsemB, ssemA, ssemB):
        wid = lax.axis_index("c") * NS + lax.axis_index("s")
        base = wid * TPW
        iota = lax.iota(jnp.int32, L)
        slots = (
            dict(idx=idxA, bias=biasA, stg=stgA, dst=dstA,
                 gsem=gsemA, ssem=ssemA),
            dict(idx=idxB, bias=biasB, stg=stgB, dst=dstB,
                 gsem=gsemB, ssem=ssemB),
        )

        def issue_idx_fetch(chunk):
            r0 = base + chunk * K
            pltpu.async_copy(sp_hbm.at[pl.ds(r0, K)], sp, isem)
            pltpu.async_copy(et_hbm.at[pl.ds(r0, K)], et, isem)

        def transpose_tasks(ot, lo, hi):
            def tk(kk, _):
                for cc in range(N // L):
                    rowv = kk * N + cc * L + iota
                    for h in range(H):
                        hv = jnp.full((L,), h, jnp.int32)
                        vals = plsc.load_gather(ot["bias"], [rowv, hv])
                        ot["stg"][2 * (h * K + kk) + cc // 8,
                                  pl.ds((cc % 8) * L, L)] = vals
                return ()
            lax.fori_loop(lo, hi, tk, ())

        def half_iter(c, par):
            me, ot = slots[par], slots[1 - par]
            in_front = c < CPT
            in_back = jnp.logical_and(c >= 1, c <= CPT)

            @pl.when(in_front)
            def _front():
                r0 = base + c * K
                pltpu.make_async_copy(sp_hbm.at[pl.ds(r0, K)], sp,
                                      isem).wait()
                pltpu.make_async_copy(et_hbm.at[pl.ds(r0, K)], et,
                                      isem).wait()

                def dk(kk, _):
                    for d in range(EDGE_DIM):
                        for cc in range(16):
                            vals = plsc.load_gather(
                                et.at[kk],
                                [iota * EDGE_DIM + (cc * 64 + d)])
                            me["idx"][d, pl.ds(kk * N + cc * L, L)] = (
                                vals + (NUM_SPATIAL + 1))
                    for cc in range(16):
                        me["idx"][EDGE_DIM, pl.ds(kk * N + cc * L, L)] = (
                            sp[kk, pl.ds(cc * L, L)])
                    return ()
                lax.fori_loop(0, K, dk, ())

            @pl.when(in_back)
            def _drain_prev():
                for d in range(EDGE_DIM):
                    pltpu.make_async_copy(ct_hbm.at[ot["idx"].at[d]],
                                          ot["bias"], ot["gsem"]).wait()
                @pl.when(c >= 3)
                def _():
                    pltpu.make_async_copy(ot["stg"], out_hbm.at[ot["dst"]],
                                          ot["ssem"]).wait()

            @pl.when(in_front)
            def _spatial():
                pltpu.async_copy(ct_hbm.at[me["idx"].at[EDGE_DIM]],
                                 me["bias"], me["gsem"])

                @pl.when(c + 1 < CPT)
                def _():
                    issue_idx_fetch(c + 1)

            @pl.when(in_back)
            def _back1():
                transpose_tasks(ot, 0, K // 2)

            @pl.when(in_front)
            def _edges():
                pltpu.make_async_copy(ct_hbm.at[me["idx"].at[EDGE_DIM]],
                                      me["bias"], me["gsem"]).wait()
                for d in range(EDGE_DIM):
                    pltpu.async_copy(ct_hbm.at[me["idx"].at[d]], me["bias"],
                                     me["gsem"], add=True)

            @pl.when(in_back)
            def _back2():
                transpose_tasks(ot, K // 2, K)
                r0 = base + (c - 1) * K
                bb = r0 // N
                i0 = r0 - bb * N
                cbase = bb * (H * N) + i0
                for kk in range(K):
                    vec = (iota * N + (cbase + kk)) * 2
                    plsc.store_scatter(ot["dst"], [iota * (2 * K) + 2 * kk],
                                       vec)
                    plsc.store_scatter(ot["dst"],
                                       [iota * (2 * K) + (2 * kk + 1)],
                                       vec + 1)
                pltpu.async_copy(ot["stg"], out_hbm.at[ot["dst"]], ot["ssem"])

        issue_idx_fetch(0)

        def pair(c2, _):
            half_iter(2 * c2, 0)
            half_iter(2 * c2 + 1, 1)
            return ()
        lax.fori_loop(0, CPT // 2 + 1, pair, ())

        for sl in slots:
            pltpu.make_async_copy(sl["stg"], out_hbm.at[sl["dst"]],
                                  sl["ssem"]).wait()

    return k(ctable, sp1, et1)


def _asm_body(ab_ref, e_ref, t_ref, o_ref):
    ab2 = ab_ref[0] * 2.0
    for h in range(H):
        tv = t_ref[0, h]
        e = e_ref[pl.ds(h * 2 * N, 2 * N), :].reshape(N, N)
        o_ref[0, h, 0:1, :] = ab2[0:1, :] + tv
        o_ref[0, h, 1:, 0:1] = ab2[1:, 0:1] + tv
        o_ref[0, h, 1:, 1:] = ab2[1:, 1:] + e


def _tc_assemble(ab, et4, t, *, interpret=False):
    return pl.pallas_call(
        _asm_body,
        grid=(B,),
        in_specs=[
            pl.BlockSpec((1, N1, N1), lambda b: (b, 0, 0)),
            pl.BlockSpec((H * 2 * N, N // 2), lambda b: (b, 0)),
            pl.BlockSpec(memory_space=pltpu.SMEM),
        ],
        out_specs=pl.BlockSpec((1, H, N1, N1), lambda b: (b, 0, 0, 0)),
        out_shape=jax.ShapeDtypeStruct((B, H, N1, N1), jnp.float32),
        interpret=interpret,
    )(ab, et4, t)


def kernel(attn_bias, spatial_pos, edge_input, attn_edge_type,
           edge_encoder_weight, spatial_pos_encoder_weight,
           graph_token_virtual_distance_weight):
    sw0 = spatial_pos_encoder_weight.at[0].set(0.0)
    ew0 = edge_encoder_weight.at[0].set(0.0) * 0.25
    ctable = jnp.concatenate([sw0, ew0], axis=0)
    sp2 = spatial_pos.reshape(B * N, N)
    et2 = attn_edge_type.reshape(B * N, N * EDGE_DIM)
    et_flat = _sc_bias(ctable, sp2, et2)
    return _tc_assemble(attn_bias, et_flat,
                        graph_token_virtual_distance_weight)
